# baseline (speedup 1.0000x reference)
import jax
import jax.numpy as jnp
from jax.experimental import pallas as pl

_FOLD = 4


def _gates_kernel(x_ref, wi_ref, wc_ref, wo_ref, bi_ref, bc_ref, bo_ref,
                  wco_ref, wlin_ref, blin_ref, o_ref):
    x = x_ref[...]
    i = jnp.tanh(
        jnp.dot(x, wi_ref[...], preferred_element_type=jnp.float32)
        + bi_ref[...]) * 0.5 + 0.5
    t = jnp.tanh(
        jnp.dot(x, wc_ref[...], preferred_element_type=jnp.float32)
        + bc_ref[...])
    c = i * t
    o = jnp.tanh(
        jnp.dot(x, wo_ref[...], preferred_element_type=jnp.float32)
        + bo_ref[...] + wco_ref[...] * c) * 0.5 + 0.5
    h = jnp.maximum(o * jnp.tanh(c), 0.0)
    o_ref[...] = (
        jnp.dot(h, wlin_ref[...], preferred_element_type=jnp.float32)
        + blin_ref[...])


def _blockdiag(w, fold):
    f_in, f_out = w.shape
    return jnp.concatenate(
        [jnp.pad(w, ((0, 0), (f_out * j, f_out * (fold - 1 - j))))
         for j in range(fold)], axis=0)


def kernel(x, edge_index, edge_weight, W_x_i, b_x_i, W_h_i, b_h_i, b_i,
           W_x_f, b_x_f, W_h_f, b_h_f, b_f, W_x_c, b_x_c, W_h_c, b_h_c, b_c,
           W_x_o, b_x_o, W_h_o, b_h_o, b_o, w_c_i, w_c_f, w_c_o, W_lin, b_lin):
    n, f_in = x.shape
    f_out = W_x_i.shape[1]
    fold = _FOLD
    nf = n // fold

    xr = x.reshape(nf, fold * f_in)
    wi = _blockdiag(W_x_i * 0.5, fold)
    wc = _blockdiag(W_x_c, fold)
    wo = _blockdiag(W_x_o * 0.5, fold)
    wlin = _blockdiag(W_lin, fold)
    bi = jnp.tile((b_x_i + b_h_i).reshape(1, f_out) + b_i, (1, fold)) * 0.5
    bc = jnp.tile((b_x_c + b_h_c).reshape(1, f_out) + b_c, (1, fold))
    bo = jnp.tile((b_x_o + b_h_o).reshape(1, f_out) + b_o, (1, fold)) * 0.5
    wco = jnp.tile(w_c_o, (1, fold)) * 0.5
    blin = jnp.broadcast_to(b_lin.reshape(1, 1), (1, fold))

    full = lambda shape: pl.BlockSpec(shape, lambda i: (0, 0))
    out = pl.pallas_call(
        _gates_kernel,
        grid=(1,),
        in_specs=[
            full((nf, fold * f_in)),
            full((fold * f_in, fold * f_out)),
            full((fold * f_in, fold * f_out)),
            full((fold * f_in, fold * f_out)),
            full((1, fold * f_out)), full((1, fold * f_out)),
            full((1, fold * f_out)), full((1, fold * f_out)),
            full((fold * f_out, fold)), full((1, fold)),
        ],
        out_specs=full((nf, fold)),
        out_shape=jax.ShapeDtypeStruct((nf, fold), jnp.float32),
    )(xr, wi, wc, wo, bi, bc, bo, wco, wlin, blin)
    return out.reshape(n, 1)


# speedup vs baseline: 1.2926x; 1.2926x over previous
import jax
import jax.numpy as jnp
from jax.experimental import pallas as pl

_BLOCK = 2000


def _gates_kernel(x_ref, wi_ref, wc_ref, wo_ref, bi_ref, bc_ref, bo_ref,
                  wco_ref, wlin_ref, blin_ref, o_ref):
    x = x_ref[...]
    f32 = jnp.float32
    zi = jnp.dot(x, wi_ref[...], preferred_element_type=f32).T
    zc = jnp.dot(x, wc_ref[...], preferred_element_type=f32).T
    zo = jnp.dot(x, wo_ref[...], preferred_element_type=f32).T
    bi = bi_ref[...].T
    bc = bc_ref[...].T
    bo = bo_ref[...].T
    wco = wco_ref[...].T
    i = jnp.tanh(zi * 0.5 + bi) * 0.5 + 0.5
    t = jnp.tanh(zc + bc)
    c = i * t
    o = jnp.tanh(zo * 0.5 + bo + wco * c) * 0.5 + 0.5
    h = jnp.maximum(o * jnp.tanh(c), 0.0)
    row = jnp.dot(wlin_ref[...], h, preferred_element_type=f32)
    o_ref[...] = row.T + blin_ref[...]


def kernel(x, edge_index, edge_weight, W_x_i, b_x_i, W_h_i, b_h_i, b_i,
           W_x_f, b_x_f, W_h_f, b_h_f, b_f, W_x_c, b_x_c, W_h_c, b_h_c, b_c,
           W_x_o, b_x_o, W_h_o, b_h_o, b_o, w_c_i, w_c_f, w_c_o, W_lin, b_lin):
    n, f_in = x.shape
    f_out = W_x_i.shape[1]

    bi = ((b_x_i + b_h_i).reshape(1, f_out) + b_i) * 0.5
    bc = (b_x_c + b_h_c).reshape(1, f_out) + b_c
    bo = ((b_x_o + b_h_o).reshape(1, f_out) + b_o) * 0.5
    wco = w_c_o * 0.5
    wlin = W_lin.reshape(1, f_out)
    blin = b_lin.reshape(1, 1)

    full = lambda shape: pl.BlockSpec(shape, lambda i: (0, 0))
    return pl.pallas_call(
        _gates_kernel,
        grid=(n // _BLOCK,),
        in_specs=[
            pl.BlockSpec((_BLOCK, f_in), lambda i: (i, 0)),
            full((f_in, f_out)), full((f_in, f_out)), full((f_in, f_out)),
            full((1, f_out)), full((1, f_out)), full((1, f_out)),
            full((1, f_out)), full((1, f_out)), full((1, 1)),
        ],
        out_specs=pl.BlockSpec((_BLOCK, 1), lambda i: (i, 0)),
        out_shape=jax.ShapeDtypeStruct((n, 1), jnp.float32),
    )(x, W_x_i, W_x_c, W_x_o, bi, bc, bo, wco, wlin, blin)


# speedup vs baseline: 1.4597x; 1.1293x over previous
import jax
import jax.numpy as jnp
from jax.experimental import pallas as pl

_BLOCK = 2048


def _gates_kernel(x_ref, wi_ref, wc_ref, wo_ref, bi_ref, bc_ref, bo_ref,
                  wco_ref, wlin_ref, blin_ref, o_ref):
    x = x_ref[...]
    f32 = jnp.float32
    zi = jnp.dot(x, wi_ref[...], preferred_element_type=f32).T
    zc = jnp.dot(x, wc_ref[...], preferred_element_type=f32).T
    zo = jnp.dot(x, wo_ref[...], preferred_element_type=f32).T
    bi = bi_ref[...].T
    bc = bc_ref[...].T
    bo = bo_ref[...].T
    wco = wco_ref[...].T
    i = jnp.tanh(zi * 0.5 + bi) * 0.5 + 0.5
    t = jnp.tanh(zc + bc)
    c = i * t
    o = jnp.tanh(zo * 0.5 + bo + wco * c) * 0.5 + 0.5
    h = jnp.maximum(o * jnp.tanh(c), 0.0)
    row = jnp.dot(wlin_ref[...], h, preferred_element_type=f32)
    o_ref[...] = row + blin_ref[...]


def kernel(x, edge_index, edge_weight, W_x_i, b_x_i, W_h_i, b_h_i, b_i,
           W_x_f, b_x_f, W_h_f, b_h_f, b_f, W_x_c, b_x_c, W_h_c, b_h_c, b_c,
           W_x_o, b_x_o, W_h_o, b_h_o, b_o, w_c_i, w_c_f, w_c_o, W_lin, b_lin):
    n, f_in = x.shape
    f_out = W_x_i.shape[1]

    bi = ((b_x_i + b_h_i).reshape(1, f_out) + b_i) * 0.5
    bc = (b_x_c + b_h_c).reshape(1, f_out) + b_c
    bo = ((b_x_o + b_h_o).reshape(1, f_out) + b_o) * 0.5
    wco = w_c_o * 0.5
    wlin = W_lin.reshape(1, f_out)
    blin = b_lin.reshape(1, 1)

    full = lambda shape: pl.BlockSpec(shape, lambda i: (0, 0))
    return pl.pallas_call(
        _gates_kernel,
        grid=(pl.cdiv(n, _BLOCK),),
        in_specs=[
            pl.BlockSpec((_BLOCK, f_in), lambda i: (i, 0)),
            full((f_in, f_out)), full((f_in, f_out)), full((f_in, f_out)),
            full((1, f_out)), full((1, f_out)), full((1, f_out)),
            full((1, f_out)), full((1, f_out)), full((1, 1)),
        ],
        out_specs=pl.BlockSpec((1, _BLOCK), lambda i: (0, i)),
        out_shape=jax.ShapeDtypeStruct((1, n), jnp.float32),
    )(x, W_x_i, W_x_c, W_x_o, bi, bc, bo, wco, wlin, blin).reshape(n, 1)


# speedup vs baseline: 2.3363x; 1.6005x over previous
import jax
import jax.numpy as jnp
from jax.experimental import pallas as pl

_BLOCK = 2048


def _gates_kernel(x_ref, wi_ref, wc_ref, wo_ref, bxi_ref, bhi_ref, bi_ref,
                  bxc_ref, bhc_ref, bc_ref, bxo_ref, bho_ref, bo_ref,
                  wco_ref, wlin_ref, blin_ref, o_ref):
    f32 = jnp.float32
    xT = x_ref[...].T
    zi = jnp.dot(wi_ref[...].T, xT, preferred_element_type=f32)
    zc = jnp.dot(wc_ref[...].T, xT, preferred_element_type=f32)
    zo = jnp.dot(wo_ref[...].T, xT, preferred_element_type=f32)
    bi = ((bxi_ref[...] + bhi_ref[...] + bi_ref[...]) * 0.5).T
    bc = (bxc_ref[...] + bhc_ref[...] + bc_ref[...]).T
    bo = ((bxo_ref[...] + bho_ref[...] + bo_ref[...]) * 0.5).T
    wco = (wco_ref[...] * 0.5).T
    i = jnp.tanh(zi * 0.5 + bi) * 0.5 + 0.5
    t = jnp.tanh(zc + bc)
    c = i * t
    o = jnp.tanh(zo * 0.5 + bo + wco * c) * 0.5 + 0.5
    h = jnp.maximum(o * jnp.tanh(c), 0.0)
    row = jnp.dot(wlin_ref[...], h, preferred_element_type=f32)
    o_ref[...] = row + blin_ref[...]


def kernel(x, edge_index, edge_weight, W_x_i, b_x_i, W_h_i, b_h_i, b_i,
           W_x_f, b_x_f, W_h_f, b_h_f, b_f, W_x_c, b_x_c, W_h_c, b_h_c, b_c,
           W_x_o, b_x_o, W_h_o, b_h_o, b_o, w_c_i, w_c_f, w_c_o, W_lin, b_lin):
    n, f_in = x.shape
    f_out = W_x_i.shape[1]

    r = lambda b: b.reshape(1, f_out)
    full = lambda shape: pl.BlockSpec(shape, lambda i: (0, 0))
    out = pl.pallas_call(
        _gates_kernel,
        grid=(pl.cdiv(n, _BLOCK),),
        in_specs=[
            pl.BlockSpec((_BLOCK, f_in), lambda i: (i, 0)),
            full((f_in, f_out)), full((f_in, f_out)), full((f_in, f_out)),
            full((1, f_out)), full((1, f_out)), full((1, f_out)),
            full((1, f_out)), full((1, f_out)), full((1, f_out)),
            full((1, f_out)), full((1, f_out)), full((1, f_out)),
            full((1, f_out)), full((1, f_out)), full((1, 1)),
        ],
        out_specs=pl.BlockSpec((1, _BLOCK), lambda i: (0, i)),
        out_shape=jax.ShapeDtypeStruct((1, n), jnp.float32),
    )(x, W_x_i, W_x_c, W_x_o,
      r(b_x_i), r(b_h_i), b_i, r(b_x_c), r(b_h_c), b_c,
      r(b_x_o), r(b_h_o), b_o, w_c_o, W_lin.reshape(1, f_out),
      b_lin.reshape(1, 1))
    return out.reshape(n, 1)


# speedup vs baseline: 2.4431x; 1.0457x over previous
import jax
import jax.numpy as jnp
from jax.experimental import pallas as pl

_BLOCK = 4096


def _gates_kernel(x_ref, wi_ref, wc_ref, wo_ref, bxi_ref, bhi_ref, bi_ref,
                  bxc_ref, bhc_ref, bc_ref, bxo_ref, bho_ref, bo_ref,
                  wco_ref, wlin_ref, blin_ref, o_ref):
    f32 = jnp.float32
    xT = x_ref[...].T
    zi = jnp.dot(wi_ref[...].T, xT, preferred_element_type=f32)
    zc = jnp.dot(wc_ref[...].T, xT, preferred_element_type=f32)
    zo = jnp.dot(wo_ref[...].T, xT, preferred_element_type=f32)
    bi = ((bxi_ref[...] + bhi_ref[...] + bi_ref[...]) * 0.5).T
    bc = (bxc_ref[...] + bhc_ref[...] + bc_ref[...]).T
    bo = ((bxo_ref[...] + bho_ref[...] + bo_ref[...]) * 0.5).T
    wco = (wco_ref[...] * 0.5).T
    i = jnp.tanh(zi * 0.5 + bi) * 0.5 + 0.5
    t = jnp.tanh(zc + bc)
    c = i * t
    o = jnp.tanh(zo * 0.5 + bo + wco * c) * 0.5 + 0.5
    h = jnp.maximum(o * jnp.tanh(c), 0.0)
    row = jnp.dot(wlin_ref[...], h, preferred_element_type=f32)
    o_ref[...] = row + blin_ref[...]


def kernel(x, edge_index, edge_weight, W_x_i, b_x_i, W_h_i, b_h_i, b_i,
           W_x_f, b_x_f, W_h_f, b_h_f, b_f, W_x_c, b_x_c, W_h_c, b_h_c, b_c,
           W_x_o, b_x_o, W_h_o, b_h_o, b_o, w_c_i, w_c_f, w_c_o, W_lin, b_lin):
    n, f_in = x.shape
    f_out = W_x_i.shape[1]

    r = lambda b: b.reshape(1, f_out)
    full = lambda shape: pl.BlockSpec(shape, lambda i: (0, 0))
    out = pl.pallas_call(
        _gates_kernel,
        grid=(pl.cdiv(n, _BLOCK),),
        in_specs=[
            pl.BlockSpec((_BLOCK, f_in), lambda i: (i, 0)),
            full((f_in, f_out)), full((f_in, f_out)), full((f_in, f_out)),
            full((1, f_out)), full((1, f_out)), full((1, f_out)),
            full((1, f_out)), full((1, f_out)), full((1, f_out)),
            full((1, f_out)), full((1, f_out)), full((1, f_out)),
            full((1, f_out)), full((1, f_out)), full((1, 1)),
        ],
        out_specs=pl.BlockSpec((1, _BLOCK), lambda i: (0, i)),
        out_shape=jax.ShapeDtypeStruct((1, n), jnp.float32),
    )(x, W_x_i, W_x_c, W_x_o,
      r(b_x_i), r(b_h_i), b_i, r(b_x_c), r(b_h_c), b_c,
      r(b_x_o), r(b_h_o), b_o, w_c_o, W_lin.reshape(1, f_out),
      b_lin.reshape(1, 1))
    return out.reshape(n, 1)


# speedup vs baseline: 2.5805x; 1.0562x over previous
import jax
import jax.numpy as jnp
from jax.experimental import pallas as pl

_BLOCK = 5120


def _gates_kernel(x_ref, wi_ref, wc_ref, wo_ref, bxi_ref, bhi_ref, bi_ref,
                  bxc_ref, bhc_ref, bc_ref, bxo_ref, bho_ref, bo_ref,
                  wco_ref, wlin_ref, blin_ref, o_ref):
    f32 = jnp.float32
    xT = x_ref[...].T
    zi = jnp.dot(wi_ref[...].T, xT, preferred_element_type=f32)
    zc = jnp.dot(wc_ref[...].T, xT, preferred_element_type=f32)
    zo = jnp.dot(wo_ref[...].T, xT, preferred_element_type=f32)
    bi = ((bxi_ref[...] + bhi_ref[...] + bi_ref[...]) * 0.5).T
    bc = (bxc_ref[...] + bhc_ref[...] + bc_ref[...]).T
    bo = ((bxo_ref[...] + bho_ref[...] + bo_ref[...]) * 0.5).T
    wco = (wco_ref[...] * 0.5).T
    i = jnp.tanh(zi * 0.5 + bi) * 0.5 + 0.5
    t = jnp.tanh(zc + bc)
    c = i * t
    o = jnp.tanh(zo * 0.5 + bo + wco * c) * 0.5 + 0.5
    h = jnp.maximum(o * jnp.tanh(c), 0.0)
    row = jnp.dot(wlin_ref[...], h, preferred_element_type=f32)
    o_ref[...] = row + blin_ref[...]


def kernel(x, edge_index, edge_weight, W_x_i, b_x_i, W_h_i, b_h_i, b_i,
           W_x_f, b_x_f, W_h_f, b_h_f, b_f, W_x_c, b_x_c, W_h_c, b_h_c, b_c,
           W_x_o, b_x_o, W_h_o, b_h_o, b_o, w_c_i, w_c_f, w_c_o, W_lin, b_lin):
    n, f_in = x.shape
    f_out = W_x_i.shape[1]

    r = lambda b: b.reshape(1, f_out)
    full = lambda shape: pl.BlockSpec(shape, lambda i: (0, 0))
    out = pl.pallas_call(
        _gates_kernel,
        grid=(pl.cdiv(n, _BLOCK),),
        in_specs=[
            pl.BlockSpec((_BLOCK, f_in), lambda i: (i, 0)),
            full((f_in, f_out)), full((f_in, f_out)), full((f_in, f_out)),
            full((1, f_out)), full((1, f_out)), full((1, f_out)),
            full((1, f_out)), full((1, f_out)), full((1, f_out)),
            full((1, f_out)), full((1, f_out)), full((1, f_out)),
            full((1, f_out)), full((1, f_out)), full((1, 1)),
        ],
        out_specs=pl.BlockSpec((1, _BLOCK), lambda i: (0, i)),
        out_shape=jax.ShapeDtypeStruct((1, n), jnp.float32),
    )(x, W_x_i, W_x_c, W_x_o,
      r(b_x_i), r(b_h_i), b_i, r(b_x_c), r(b_h_c), b_c,
      r(b_x_o), r(b_h_o), b_o, w_c_o, W_lin.reshape(1, f_out),
      b_lin.reshape(1, 1))
    return out.reshape(n, 1)
